# baseline (speedup 1.0000x reference)
import functools

import jax
import jax.numpy as jnp
from jax import lax
from jax.experimental import pallas as pl
from jax.experimental.pallas import tpu as pltpu
from jax.experimental.pallas import tpu_sc as plsc

B = 16384
GMF_DIM = 32
MLP_DIM = 128

NC = 2
NS = 16
NW = NC * NS
BPW = B // NW
CH = 128
NCH = BPW // CH

_sc_mesh = plsc.VectorSubcoreMesh(
    core_axis_name="c", subcore_axis_name="s", num_cores=NC, num_subcores=NS
)


@functools.partial(
    pl.kernel,
    out_type=(
        jax.ShapeDtypeStruct((B, GMF_DIM), jnp.float32),
        jax.ShapeDtypeStruct((B, GMF_DIM), jnp.float32),
        jax.ShapeDtypeStruct((B, MLP_DIM), jnp.float32),
        jax.ShapeDtypeStruct((B, MLP_DIM), jnp.float32),
    ),
    mesh=_sc_mesh,
    scratch_types=(
        pltpu.VMEM((NCH, CH), jnp.int32),
        pltpu.VMEM((NCH, CH), jnp.int32),
        pltpu.VMEM((CH, GMF_DIM), jnp.float32),
        pltpu.VMEM((CH, GMF_DIM), jnp.float32),
        pltpu.VMEM((CH, MLP_DIM), jnp.float32),
        pltpu.VMEM((CH, MLP_DIM), jnp.float32),
        pltpu.SemaphoreType.DMA,
    ),
    compiler_params=pltpu.CompilerParams(use_tc_tiling_on_sc=False),
)
def _gather_sc(ui_hbm, ii_hbm, gmf_p_hbm, gmf_q_hbm, mlp_p_hbm, mlp_q_hbm,
               gu_out, gi_out, mu_out, mq_out,
               ui_v, ii_v, gu_v, gi_v, mu_v, mq_v, sem):
    wid = lax.axis_index("s") * NC + lax.axis_index("c")
    base = wid * BPW
    for c in range(NCH):
        pltpu.sync_copy(ui_hbm.at[pl.ds(base + c * CH, CH)], ui_v.at[c])
        pltpu.sync_copy(ii_hbm.at[pl.ds(base + c * CH, CH)], ii_v.at[c])
    for c in range(NCH):
        cp1 = pltpu.async_copy(gmf_p_hbm.at[ui_v.at[c]], gu_v, sem)
        cp2 = pltpu.async_copy(gmf_q_hbm.at[ii_v.at[c]], gi_v, sem)
        cp3 = pltpu.async_copy(mlp_p_hbm.at[ui_v.at[c]], mu_v, sem)
        cp4 = pltpu.async_copy(mlp_q_hbm.at[ii_v.at[c]], mq_v, sem)
        cp1.wait()
        cp2.wait()
        cp3.wait()
        cp4.wait()
        off = base + c * CH
        pltpu.sync_copy(gu_v, gu_out.at[pl.ds(off, CH)])
        pltpu.sync_copy(gi_v, gi_out.at[pl.ds(off, CH)])
        pltpu.sync_copy(mu_v, mu_out.at[pl.ds(off, CH)])
        pltpu.sync_copy(mq_v, mq_out.at[pl.ds(off, CH)])


_BB = 1024


def _mlp_body(gu_ref, gi_ref, mu_ref, mq_ref,
              w1_ref, b1_ref, w2_ref, b2_ref, w3_ref, b3_ref,
              wo_ref, bo_ref, out_ref):
    h = jnp.dot(mu_ref[...], w1_ref[0:MLP_DIM, :], preferred_element_type=jnp.float32)
    h = h + jnp.dot(mq_ref[...], w1_ref[MLP_DIM:2 * MLP_DIM, :],
                    preferred_element_type=jnp.float32)
    h = jnp.maximum(h + b1_ref[...], 0.0)
    h = jnp.maximum(jnp.dot(h, w2_ref[...], preferred_element_type=jnp.float32)
                    + b2_ref[...], 0.0)
    h = jnp.maximum(jnp.dot(h, w3_ref[...], preferred_element_type=jnp.float32)
                    + b3_ref[...], 0.0)
    g = gu_ref[...] * gi_ref[...]
    out = (jnp.dot(g, wo_ref[0:GMF_DIM, :], preferred_element_type=jnp.float32)
           + jnp.dot(h, wo_ref[GMF_DIM:2 * GMF_DIM, :],
                     preferred_element_type=jnp.float32)
           + bo_ref[...])
    out_ref[...] = out


def _mlp_tc(gu, gi, mu, mq, w1, b1, w2, b2, w3, b3, wo, bo):
    grid = B // _BB
    blk = lambda r, c: pl.BlockSpec((r, c), lambda i: (i, 0))
    full = lambda r, c: pl.BlockSpec((r, c), lambda i: (0, 0))
    return pl.pallas_call(
        _mlp_body,
        grid=(grid,),
        in_specs=[
            blk(_BB, GMF_DIM), blk(_BB, GMF_DIM),
            blk(_BB, MLP_DIM), blk(_BB, MLP_DIM),
            full(256, 128), full(1, 128),
            full(128, 64), full(1, 64),
            full(64, 32), full(1, 32),
            full(64, 1), full(1, 1),
        ],
        out_specs=blk(_BB, 1),
        out_shape=jax.ShapeDtypeStruct((B, 1), jnp.float32),
    )(gu, gi, mu, mq, w1, b1, w2, b2, w3, b3, wo, bo)


def kernel(user_id, item_id, gmf_P, gmf_Q, mlp_P, mlp_Q,
           W1, b1, W2, b2, W3, b3, Wout, bout):
    ui = user_id - 1
    ii = item_id - 1
    gu, gi, mu, mq = _gather_sc(ui, ii, gmf_P, gmf_Q, mlp_P, mlp_Q)
    return _mlp_tc(gu, gi, mu, mq,
                   W1, b1.reshape(1, -1), W2, b2.reshape(1, -1),
                   W3, b3.reshape(1, -1), Wout, bout.reshape(1, 1))
